# baseline (speedup 1.0000x reference)
import functools

import jax
import jax.numpy as jnp
from jax import lax
from jax.experimental import pallas as pl
from jax.experimental.pallas import tpu as pltpu

_CPAD = 32
_CSPLIT = 2


def _loss_body(c, h, w, f_ref, l_ref, pt_ref, out_ref, acc, m_buf):
    ch = c // _CSPLIT
    j = pl.program_id(1)
    step = pl.program_id(0) * _CSPLIT + j
    nsteps = pl.num_programs(0) * _CSPLIT

    @pl.when(step == 0)
    def _init():
        acc[...] = jnp.zeros_like(acc)

    f = f_ref[0].reshape(ch, h * w)

    @pl.when(j == 0)
    def _mk_mask():
        labs = l_ref[0].reshape(1, h * w)
        klass = lax.broadcasted_iota(jnp.int32, (_CPAD, h * w), 0)
        m_buf[...] = (klass == labs).astype(jnp.float32)

    rowsq = jnp.sum(f * f, axis=0, keepdims=True)
    g = jnp.concatenate(
        [f, rowsq, jnp.ones_like(rowsq)], axis=0)
    out = lax.dot_general(
        g, m_buf[...], (((1,), (1,)), ((), ())),
        preferred_element_type=jnp.float32)

    @pl.when(j == 0)
    def _acc0():
        acc[0:ch, :] += out[0:ch, :]

    @pl.when(j == 1)
    def _acc1():
        acc[ch:2 * ch, :] += out[0:ch, :]

    acc[2 * ch:2 * ch + 2, :] += out[ch:ch + 2, :]

    @pl.when(step == nsteps - 1)
    def _finish():
        s = acc[0:c, :]
        q = acc[c:c + 1, :]
        n = acc[c + 1:c + 2, :] * jnp.float32(1.0 / _CSPLIT)
        pt = pt_ref[...]
        ps = jnp.sum(pt * s, axis=0, keepdims=True)
        pp = jnp.sum(pt * pt, axis=0, keepdims=True)
        present = n > 0.0
        denom = jnp.where(present, n, 1.0) * jnp.float32(c)
        term = jnp.where(present, (q + n * pp - 2.0 * ps) / denom, 0.0)
        loss = jnp.sum(term) / jnp.sum(present.astype(jnp.float32))
        out_ref[0, 0] = loss


def kernel(features, labels, prototypes):
    b, c, h, w = features.shape
    ncls = prototypes.shape[0]

    labs = labels.astype(jnp.int32).reshape(b, 1, h, w)
    pt = jnp.zeros((c, _CPAD), jnp.float32).at[:, :ncls].set(prototypes.T)

    out = pl.pallas_call(
        functools.partial(_loss_body, c, h, w),
        grid=(b, _CSPLIT),
        in_specs=[
            pl.BlockSpec((1, c // _CSPLIT, h, w), lambda i, j: (i, j, 0, 0)),
            pl.BlockSpec((1, 1, h, w), lambda i, j: (i, 0, 0, 0)),
            pl.BlockSpec((c, _CPAD), lambda i, j: (0, 0)),
        ],
        out_specs=pl.BlockSpec(memory_space=pltpu.SMEM),
        out_shape=jax.ShapeDtypeStruct((1, 1), jnp.float32),
        scratch_shapes=[
            pltpu.VMEM((c + 2, _CPAD), jnp.float32),
            pltpu.VMEM((_CPAD, h * w), jnp.float32),
        ],
    )(features, labs, pt)
    return out.reshape(())


# speedup vs baseline: 1.2377x; 1.2377x over previous
import functools

import jax
import jax.numpy as jnp
from jax import lax
from jax.experimental import pallas as pl
from jax.experimental.pallas import tpu as pltpu

_CPAD = 32


def _loss_body(c, h, w, f_ref, l_ref, pt_ref, out_ref, acc):
    step = pl.program_id(0)
    nsteps = pl.num_programs(0)

    @pl.when(step == 0)
    def _init():
        acc[...] = jnp.zeros_like(acc)

    f = f_ref[0].reshape(c, h * w)
    labs = l_ref[0].reshape(1, h * w)

    klass = lax.broadcasted_iota(jnp.int32, (_CPAD, h * w), 0)
    m = (klass == labs).astype(jnp.float32)

    rowsq = jnp.sum(f * f, axis=0, keepdims=True)
    g = jnp.concatenate(
        [f, rowsq, jnp.ones_like(rowsq)], axis=0)
    acc[...] += lax.dot_general(
        g, m, (((1,), (1,)), ((), ())),
        preferred_element_type=jnp.float32)

    @pl.when(step == nsteps - 1)
    def _finish():
        s = acc[0:c, :]
        q = acc[c:c + 1, :]
        n = acc[c + 1:c + 2, :]
        pt = pt_ref[...]
        ps = jnp.sum(pt * s, axis=0, keepdims=True)
        pp = jnp.sum(pt * pt, axis=0, keepdims=True)
        present = n > 0.0
        denom = jnp.where(present, n, 1.0) * jnp.float32(c)
        term = jnp.where(present, (q + n * pp - 2.0 * ps) / denom, 0.0)
        loss = jnp.sum(term) / jnp.sum(present.astype(jnp.float32))
        out_ref[0, 0] = loss


def kernel(features, labels, prototypes):
    b, c, h, w = features.shape
    ncls = prototypes.shape[0]

    labs = labels.astype(jnp.int32).reshape(b, 1, h, w)
    pt = jnp.zeros((c, _CPAD), jnp.float32).at[:, :ncls].set(prototypes.T)

    out = pl.pallas_call(
        functools.partial(_loss_body, c, h, w),
        grid=(b,),
        in_specs=[
            pl.BlockSpec((1, c, h, w), lambda i: (i, 0, 0, 0)),
            pl.BlockSpec((1, 1, h, w), lambda i: (i, 0, 0, 0)),
            pl.BlockSpec((c, _CPAD), lambda i: (0, 0)),
        ],
        out_specs=pl.BlockSpec(memory_space=pltpu.SMEM),
        out_shape=jax.ShapeDtypeStruct((1, 1), jnp.float32),
        scratch_shapes=[
            pltpu.VMEM((c + 2, _CPAD), jnp.float32),
        ],
    )(features, labs, pt)
    return out.reshape(())


# speedup vs baseline: 1.2654x; 1.0224x over previous
import functools

import jax
import jax.numpy as jnp
from jax import lax
from jax.experimental import pallas as pl
from jax.experimental.pallas import tpu as pltpu

_CPAD = 32


def _loss_body(c, h, w, f_ref, l_ref, f2_ref, l2_ref, pt_ref, out_ref, acc):
    step = pl.program_id(0)
    nsteps = pl.num_programs(0)

    @pl.when(step == 0)
    def _init():
        acc[...] = jnp.zeros_like(acc)

    for fr, lr in ((f_ref, l_ref), (f2_ref, l2_ref)):
        f = fr[0].reshape(c, h * w)
        labs = lr[0].reshape(1, h * w)

        klass = lax.broadcasted_iota(jnp.int32, (_CPAD, h * w), 0)
        m = (klass == labs).astype(jnp.float32)

        rowsq = jnp.sum(f * f, axis=0, keepdims=True)
        g = jnp.concatenate(
            [f, rowsq, jnp.ones_like(rowsq)], axis=0)
        acc[...] += lax.dot_general(
            g, m, (((1,), (1,)), ((), ())),
            preferred_element_type=jnp.float32)

    @pl.when(step == nsteps - 1)
    def _finish():
        s = acc[0:c, :]
        q = acc[c:c + 1, :]
        n = acc[c + 1:c + 2, :]
        pt = pt_ref[...]
        ps = jnp.sum(pt * s, axis=0, keepdims=True)
        pp = jnp.sum(pt * pt, axis=0, keepdims=True)
        present = n > 0.0
        denom = jnp.where(present, n, 1.0) * jnp.float32(c)
        term = jnp.where(present, (q + n * pp - 2.0 * ps) / denom, 0.0)
        loss = jnp.sum(term) / jnp.sum(present.astype(jnp.float32))
        out_ref[0, 0] = loss


def kernel(features, labels, prototypes):
    b, c, h, w = features.shape
    ncls = prototypes.shape[0]

    labs = labels.astype(jnp.int32).reshape(b, 1, h, w)
    pt = jnp.zeros((c, _CPAD), jnp.float32).at[:, :ncls].set(prototypes.T)

    out = pl.pallas_call(
        functools.partial(_loss_body, c, h, w),
        grid=(b // 2,),
        in_specs=[
            pl.BlockSpec((1, c, h, w), lambda i: (2 * i, 0, 0, 0)),
            pl.BlockSpec((1, 1, h, w), lambda i: (2 * i, 0, 0, 0)),
            pl.BlockSpec((1, c, h, w), lambda i: (2 * i + 1, 0, 0, 0)),
            pl.BlockSpec((1, 1, h, w), lambda i: (2 * i + 1, 0, 0, 0)),
            pl.BlockSpec((c, _CPAD), lambda i: (0, 0)),
        ],
        out_specs=pl.BlockSpec(memory_space=pltpu.SMEM),
        out_shape=jax.ShapeDtypeStruct((1, 1), jnp.float32),
        scratch_shapes=[
            pltpu.VMEM((c + 2, _CPAD), jnp.float32),
        ],
    )(features, labs, features, labs, pt)
    return out.reshape(())
